# baseline (speedup 1.0000x reference)
import jax
import jax.numpy as jnp
from jax import lax
from jax.experimental import pallas as pl
from jax.experimental.pallas import tpu as pltpu
from jax.experimental.pallas import tpu_sc as plsc

N_ROWS = 16384
D = 128
T_LEN = 1000
N_WORKERS = 32
RPW = N_ROWS // N_WORKERS
C = 128
N_CHUNKS = RPW // C
N_SLOTS = 3


def _sc_qsample(x0, noise, t, table_a, table_b, xt, noise_out,
                ta_v, tb_v, idx_v, a_v, b_v,
                xb0, xb1, xb2, nb0, nb1, nb2,
                sx0, sx1, sx2, sn0, sn1, sn2,
                so0, so1, so2, sno0, sno1, sno2):
    wid = lax.axis_index("s") * 2 + lax.axis_index("c")
    base = wid * RPW

    xbufs = [xb0, xb1, xb2]
    nbufs = [nb0, nb1, nb2]
    sxs = [sx0, sx1, sx2]
    sns = [sn0, sn1, sn2]
    sos = [so0, so1, so2]
    snos = [sno0, sno1, sno2]

    def start_in(c):
        s = c % N_SLOTS
        pltpu.async_copy(x0.at[pl.ds(base + c * C, C), :], xbufs[s], sxs[s])
        pltpu.async_copy(noise.at[pl.ds(base + c * C, C), :], nbufs[s], sns[s])

    def wait_in(c):
        s = c % N_SLOTS
        pltpu.make_async_copy(
            x0.at[pl.ds(base, C), :], xbufs[s], sxs[s]).wait()
        pltpu.make_async_copy(
            noise.at[pl.ds(base, C), :], nbufs[s], sns[s]).wait()

    def start_nout(c):
        s = c % N_SLOTS
        pltpu.async_copy(
            nbufs[s], noise_out.at[pl.ds(base + c * C, C), :], snos[s])

    def wait_nout(c):
        s = c % N_SLOTS
        pltpu.make_async_copy(
            nbufs[s], noise_out.at[pl.ds(base, C), :], snos[s]).wait()

    def start_xtout(c):
        s = c % N_SLOTS
        pltpu.async_copy(xbufs[s], xt.at[pl.ds(base + c * C, C), :], sos[s])

    def wait_xtout(c):
        s = c % N_SLOTS
        pltpu.make_async_copy(
            xbufs[s], xt.at[pl.ds(base, C), :], sos[s]).wait()

    def compute(c):
        s = c % N_SLOTS
        xbuf, nbuf = xbufs[s], nbufs[s]

        @plsc.parallel_loop(0, C, unroll=2)
        def row(r):
            ci = jnp.full((16,), c * C, jnp.int32) + r
            av = plsc.load_gather(a_v, [ci])
            bv = plsc.load_gather(b_v, [ci])
            for j in range(D // 16):
                sl = pl.ds(j * 16, 16)
                xbuf[r, sl] = av * xbuf[r, sl] + bv * nbuf[r, sl]

    start_in(0)
    start_in(1)

    pltpu.sync_copy(table_a, ta_v)
    pltpu.sync_copy(table_b, tb_v)
    pltpu.sync_copy(t.at[pl.ds(base, RPW)], idx_v)

    @plsc.parallel_loop(0, RPW // 16, unroll=2)
    def gath(j):
        iv = idx_v[pl.ds(j * 16, 16)]
        a_v[pl.ds(j * 16, 16)] = plsc.load_gather(ta_v, [iv])
        b_v[pl.ds(j * 16, 16)] = plsc.load_gather(tb_v, [iv])

    start_in(2)
    waited = set()
    for c in range(N_CHUNKS):
        wait_in(c)
        start_nout(c)
        compute(c)
        start_xtout(c)
        nxt = c + 2
        if N_SLOTS <= nxt < N_CHUNKS:
            prev = nxt - N_SLOTS
            wait_xtout(prev)
            wait_nout(prev)
            waited.add(prev)
            start_in(nxt)
    for c in range(N_CHUNKS):
        if c not in waited:
            wait_xtout(c)
            wait_nout(c)


@jax.jit
def kernel(x0, t, noise, sqrt_alpha_bar, sqrt_one_minus_alpha_bar):
    t32 = t.astype(jnp.int32)
    mesh = plsc.VectorSubcoreMesh(core_axis_name="c", subcore_axis_name="s")
    xt, noise_out = pl.kernel(
        _sc_qsample,
        out_type=(
            jax.ShapeDtypeStruct((N_ROWS, D), jnp.float32),
            jax.ShapeDtypeStruct((N_ROWS, D), jnp.float32),
        ),
        mesh=mesh,
        compiler_params=pltpu.CompilerParams(needs_layout_passes=False),
        scratch_types=[
            pltpu.VMEM((T_LEN,), jnp.float32),
            pltpu.VMEM((T_LEN,), jnp.float32),
            pltpu.VMEM((RPW,), jnp.int32),
            pltpu.VMEM((RPW,), jnp.float32),
            pltpu.VMEM((RPW,), jnp.float32),
            pltpu.VMEM((C, D), jnp.float32),
            pltpu.VMEM((C, D), jnp.float32),
            pltpu.VMEM((C, D), jnp.float32),
            pltpu.VMEM((C, D), jnp.float32),
            pltpu.VMEM((C, D), jnp.float32),
            pltpu.VMEM((C, D), jnp.float32),
            pltpu.SemaphoreType.DMA,
            pltpu.SemaphoreType.DMA,
            pltpu.SemaphoreType.DMA,
            pltpu.SemaphoreType.DMA,
            pltpu.SemaphoreType.DMA,
            pltpu.SemaphoreType.DMA,
            pltpu.SemaphoreType.DMA,
            pltpu.SemaphoreType.DMA,
            pltpu.SemaphoreType.DMA,
            pltpu.SemaphoreType.DMA,
            pltpu.SemaphoreType.DMA,
            pltpu.SemaphoreType.DMA,
        ],
    )(x0, noise, t32, sqrt_alpha_bar, sqrt_one_minus_alpha_bar)
    return (xt, noise_out)


# speedup vs baseline: 1.0083x; 1.0083x over previous
import jax
import jax.numpy as jnp
from jax import lax
from jax.experimental import pallas as pl
from jax.experimental.pallas import tpu as pltpu
from jax.experimental.pallas import tpu_sc as plsc

N_ROWS = 16384
D = 128
T_LEN = 1000
N_WORKERS = 32
RPW = N_ROWS // N_WORKERS
C = 128
N_CHUNKS = RPW // C


def _sc_qsample(x0, noise, t, table_a, table_b, xt, noise_out,
                ta_v, tb_v, idx_v, a_v, b_v,
                xb0, xb1, nb0, nb1, ob0, ob1,
                sx0, sx1, sn0, sn1, so0, so1, sno0, sno1):
    wid = lax.axis_index("s") * 2 + lax.axis_index("c")
    base = wid * RPW

    def early_in(c, s):
        pltpu.async_copy(
            x0.at[pl.ds(base + c * C, C), :],
            [xb0, xb1][s], [sx0, sx1][s])
        pltpu.async_copy(
            noise.at[pl.ds(base + c * C, C), :],
            [nb0, nb1][s], [sn0, sn1][s])

    early_in(0, 0)
    early_in(1, 1)

    pltpu.sync_copy(table_a, ta_v)
    pltpu.sync_copy(table_b, tb_v)
    pltpu.sync_copy(t.at[pl.ds(base, RPW)], idx_v)

    @plsc.parallel_loop(0, RPW // 16, unroll=2)
    def gath(j):
        iv = idx_v[pl.ds(j * 16, 16)]
        a_v[pl.ds(j * 16, 16)] = plsc.load_gather(ta_v, [iv])
        b_v[pl.ds(j * 16, 16)] = plsc.load_gather(tb_v, [iv])

    xbufs = [xb0, xb1]
    nbufs = [nb0, nb1]
    obufs = [ob0, ob1]
    sxs = [sx0, sx1]
    sns = [sn0, sn1]
    sos = [so0, so1]
    snos = [sno0, sno1]

    def start_in(c, s):
        pltpu.async_copy(x0.at[pl.ds(base + c * C, C), :], xbufs[s], sxs[s])
        pltpu.async_copy(noise.at[pl.ds(base + c * C, C), :], nbufs[s], sns[s])

    def wait_in(s):
        pltpu.make_async_copy(
            x0.at[pl.ds(base, C), :], xbufs[s], sxs[s]).wait()
        pltpu.make_async_copy(
            noise.at[pl.ds(base, C), :], nbufs[s], sns[s]).wait()

    def start_nout(c, s):
        pltpu.async_copy(
            nbufs[s], noise_out.at[pl.ds(base + c * C, C), :], snos[s])

    def wait_nout(s):
        pltpu.make_async_copy(
            nbufs[s], noise_out.at[pl.ds(base, C), :], snos[s]).wait()

    def start_xtout(c, s):
        pltpu.async_copy(obufs[s], xt.at[pl.ds(base + c * C, C), :], sos[s])

    def wait_xtout(s):
        pltpu.make_async_copy(
            obufs[s], xt.at[pl.ds(base, C), :], sos[s]).wait()

    def compute(c, s):
        xbuf, nbuf, obuf = xbufs[s], nbufs[s], obufs[s]

        @plsc.parallel_loop(0, C, unroll=2)
        def row(r):
            ci = jnp.full((16,), 0, jnp.int32) + (c * C + r)
            av = plsc.load_gather(a_v, [ci])
            bv = plsc.load_gather(b_v, [ci])
            for j in range(D // 16):
                sl = pl.ds(j * 16, 16)
                obuf[r, sl] = av * xbuf[r, sl] + bv * nbuf[r, sl]

    NPAIR = N_CHUNKS // 2

    @pl.loop(0, NPAIR)
    def pair(i):
        c0 = i * 2
        c1 = c0 + 1

        wait_in(0)
        start_nout(c0, 0)

        @pl.when(i >= 1)
        def _():
            wait_xtout(0)

        compute(c0, 0)
        start_xtout(c0, 0)

        @pl.when(i < NPAIR - 1)
        def _():
            wait_nout(0)
            start_in(c0 + 2, 0)

        wait_in(1)
        start_nout(c1, 1)

        @pl.when(i >= 1)
        def _():
            wait_xtout(1)

        compute(c1, 1)
        start_xtout(c1, 1)

        @pl.when(i < NPAIR - 1)
        def _():
            wait_nout(1)
            start_in(c1 + 2, 1)

    wait_xtout(0)
    wait_xtout(1)
    wait_nout(0)
    wait_nout(1)


@jax.jit
def kernel(x0, t, noise, sqrt_alpha_bar, sqrt_one_minus_alpha_bar):
    t32 = t.astype(jnp.int32)
    mesh = plsc.VectorSubcoreMesh(core_axis_name="c", subcore_axis_name="s")
    xt, noise_out = pl.kernel(
        _sc_qsample,
        out_type=(
            jax.ShapeDtypeStruct((N_ROWS, D), jnp.float32),
            jax.ShapeDtypeStruct((N_ROWS, D), jnp.float32),
        ),
        mesh=mesh,
        compiler_params=pltpu.CompilerParams(needs_layout_passes=False),
        scratch_types=[
            pltpu.VMEM((T_LEN,), jnp.float32),
            pltpu.VMEM((T_LEN,), jnp.float32),
            pltpu.VMEM((RPW,), jnp.int32),
            pltpu.VMEM((RPW,), jnp.float32),
            pltpu.VMEM((RPW,), jnp.float32),
            pltpu.VMEM((C, D), jnp.float32),
            pltpu.VMEM((C, D), jnp.float32),
            pltpu.VMEM((C, D), jnp.float32),
            pltpu.VMEM((C, D), jnp.float32),
            pltpu.VMEM((C, D), jnp.float32),
            pltpu.VMEM((C, D), jnp.float32),
            pltpu.SemaphoreType.DMA,
            pltpu.SemaphoreType.DMA,
            pltpu.SemaphoreType.DMA,
            pltpu.SemaphoreType.DMA,
            pltpu.SemaphoreType.DMA,
            pltpu.SemaphoreType.DMA,
            pltpu.SemaphoreType.DMA,
            pltpu.SemaphoreType.DMA,
        ],
    )(x0, noise, t32, sqrt_alpha_bar, sqrt_one_minus_alpha_bar)
    return (xt, noise_out)
